# baseline (speedup 1.0000x reference)
import jax
import jax.numpy as jnp
from jax.experimental import pallas as pl
from jax.experimental.pallas import tpu as pltpu

D_MODEL = 2048
N_FEATURE = 4096
D_SPACE = 64

TILE_M = 1024


def _normalize_kernel(emb_ref, out_ref):
    emb = emb_ref[...]
    sq = jnp.sum(emb * emb, axis=-1, keepdims=True)
    out_ref[...] = emb / jnp.maximum(jnp.sqrt(sq), 1e-12)


def _router_kernel(x_ref, w_ref, b_ref, emb_ref, out_ref):
    h = jax.lax.dot_general(
        x_ref[...], w_ref[...],
        (((1,), (1,)), ((), ())),
        preferred_element_type=jnp.float32,
    ) + b_ref[...]
    out_ref[...] = jax.lax.dot_general(
        h, emb_ref[...],
        (((1,), (1,)), ((), ())),
        preferred_element_type=jnp.float32,
    )


@jax.jit
def kernel(x, W_proj, b_proj, neuron_emb):
    B, S, _ = x.shape
    M = B * S
    x2 = x.reshape(M, D_MODEL)
    emb = neuron_emb[:N_FEATURE]
    b2 = b_proj.reshape(1, D_SPACE)

    emb_norm = pl.pallas_call(
        _normalize_kernel,
        out_shape=jax.ShapeDtypeStruct((N_FEATURE, D_SPACE), jnp.float32),
    )(emb)

    grid = (M // TILE_M,)
    out = pl.pallas_call(
        _router_kernel,
        grid=grid,
        in_specs=[
            pl.BlockSpec((TILE_M, D_MODEL), lambda m: (m, 0)),
            pl.BlockSpec((D_SPACE, D_MODEL), lambda m: (0, 0)),
            pl.BlockSpec((1, D_SPACE), lambda m: (0, 0)),
            pl.BlockSpec((N_FEATURE, D_SPACE), lambda m: (0, 0)),
        ],
        out_specs=pl.BlockSpec((TILE_M, N_FEATURE), lambda m: (m, 0)),
        out_shape=jax.ShapeDtypeStruct((M, N_FEATURE), jnp.float32),
        compiler_params=pltpu.CompilerParams(
            dimension_semantics=("parallel",),
        ),
    )(x2, W_proj, b2, emb_norm)
    return out.reshape(B, S, N_FEATURE)


# speedup vs baseline: 1.0226x; 1.0226x over previous
import jax
import jax.numpy as jnp
from jax.experimental import pallas as pl
from jax.experimental.pallas import tpu as pltpu

D_MODEL = 2048
N_FEATURE = 4096
D_SPACE = 64

TILE_M = 1024


def _router_kernel(x_ref, w_ref, b_ref, emb_ref, out_ref, emb_norm_ref):
    @pl.when(pl.program_id(0) == 0)
    def _normalize():
        emb = emb_ref[...]
        sq = jnp.sum(emb * emb, axis=-1, keepdims=True)
        emb_norm_ref[...] = (emb / jnp.maximum(jnp.sqrt(sq), 1e-12)).astype(
            jnp.bfloat16)

    h = jax.lax.dot_general(
        x_ref[...].astype(jnp.bfloat16), w_ref[...].astype(jnp.bfloat16),
        (((1,), (1,)), ((), ())),
        preferred_element_type=jnp.float32,
    ) + b_ref[...]
    out_ref[...] = jax.lax.dot_general(
        h.astype(jnp.bfloat16), emb_norm_ref[...],
        (((1,), (1,)), ((), ())),
        preferred_element_type=jnp.float32,
    )


@jax.jit
def kernel(x, W_proj, b_proj, neuron_emb):
    B, S, _ = x.shape
    M = B * S
    x2 = x.reshape(M, D_MODEL)
    emb = neuron_emb[:N_FEATURE]
    b2 = b_proj.reshape(1, D_SPACE)

    grid = (M // TILE_M,)
    out = pl.pallas_call(
        _router_kernel,
        grid=grid,
        in_specs=[
            pl.BlockSpec((TILE_M, D_MODEL), lambda m: (m, 0)),
            pl.BlockSpec((D_SPACE, D_MODEL), lambda m: (0, 0)),
            pl.BlockSpec((1, D_SPACE), lambda m: (0, 0)),
            pl.BlockSpec((N_FEATURE, D_SPACE), lambda m: (0, 0)),
        ],
        out_specs=pl.BlockSpec((TILE_M, N_FEATURE), lambda m: (m, 0)),
        out_shape=jax.ShapeDtypeStruct((M, N_FEATURE), jnp.float32),
        scratch_shapes=[pltpu.VMEM((N_FEATURE, D_SPACE), jnp.bfloat16)],
        compiler_params=pltpu.CompilerParams(
            dimension_semantics=("arbitrary",),
        ),
    )(x2, W_proj, b2, emb)
    return out.reshape(B, S, N_FEATURE)


# speedup vs baseline: 1.0281x; 1.0053x over previous
import jax
import jax.numpy as jnp
from jax.experimental import pallas as pl
from jax.experimental.pallas import tpu as pltpu

D_MODEL = 2048
N_FEATURE = 4096
D_SPACE = 64

TILE_M = 1024


def _router_kernel(x_ref, w_ref, b_ref, emb_ref, out_ref):
    emb = emb_ref[...]
    sq = jnp.sum(emb * emb, axis=-1, keepdims=True)
    emb_norm = emb / jnp.maximum(jnp.sqrt(sq), 1e-12)

    h = jax.lax.dot_general(
        x_ref[...], w_ref[...],
        (((1,), (1,)), ((), ())),
        preferred_element_type=jnp.float32,
    ) + b_ref[...]
    out_ref[...] = jax.lax.dot_general(
        h, emb_norm,
        (((1,), (1,)), ((), ())),
        preferred_element_type=jnp.float32,
    )


@jax.jit
def kernel(x, W_proj, b_proj, neuron_emb):
    B, S, _ = x.shape
    M = B * S
    x2 = x.reshape(M, D_MODEL)
    emb = neuron_emb[:N_FEATURE]
    b2 = b_proj.reshape(1, D_SPACE)

    grid = (M // TILE_M,)
    out = pl.pallas_call(
        _router_kernel,
        grid=grid,
        in_specs=[
            pl.BlockSpec((TILE_M, D_MODEL), lambda m: (m, 0)),
            pl.BlockSpec((D_SPACE, D_MODEL), lambda m: (0, 0)),
            pl.BlockSpec((1, D_SPACE), lambda m: (0, 0)),
            pl.BlockSpec((N_FEATURE, D_SPACE), lambda m: (0, 0)),
        ],
        out_specs=pl.BlockSpec((TILE_M, N_FEATURE), lambda m: (m, 0)),
        out_shape=jax.ShapeDtypeStruct((M, N_FEATURE), jnp.float32),
        compiler_params=pltpu.CompilerParams(
            dimension_semantics=("parallel",),
        ),
    )(x2, W_proj, b2, emb)
    return out.reshape(B, S, N_FEATURE)
